# baseline (speedup 1.0000x reference)
import functools

import jax
import jax.numpy as jnp
from jax import lax
from jax.experimental import pallas as pl
from jax.experimental.pallas import tpu as pltpu
from jax.experimental.pallas import tpu_sc as plsc

LANES = 16
G = 128


def _emb_sc(table, idx2, B, D, scale):
    info = plsc.get_sparse_core_info()
    NC, NS = info.num_cores, info.num_subcores
    NW = NC * NS
    n_groups = idx2.shape[0]
    rows_per_w = n_groups // NW

    mesh = plsc.VectorSubcoreMesh(core_axis_name="c", subcore_axis_name="s")

    @functools.partial(
        pl.kernel,
        mesh=mesh,
        compiler_params=pltpu.CompilerParams(use_tc_tiling_on_sc=False),
        out_type=jax.ShapeDtypeStruct((B, D), jnp.float32),
        scratch_types=[
            pltpu.VMEM((rows_per_w, G), jnp.int32),
            pltpu.VMEM((G, D), jnp.float32),
            pltpu.SemaphoreType.DMA,
        ],
    )
    def emb(table_hbm, idx_hbm, out_hbm, idx_v, rows_v, sem):
        c = lax.axis_index("c")
        s = lax.axis_index("s")
        wid = s * NC + c
        gbase = wid * rows_per_w
        pltpu.sync_copy(idx_hbm.at[pl.ds(gbase, rows_per_w)], idx_v)

        def group_body(i, carry):
            pltpu.async_copy(table_hbm.at[idx_v.at[i]], rows_v, sem).wait()

            def scale_row(r, carry2):
                for j in range(D // LANES):
                    sl = pl.ds(j * LANES, LANES)
                    rows_v[r, sl] = rows_v[r, sl] * scale
                return carry2

            lax.fori_loop(0, G, scale_row, 0)
            pltpu.sync_copy(rows_v, out_hbm.at[pl.ds((gbase + i) * G, G)])
            return carry

        lax.fori_loop(0, rows_per_w, group_body, 0)

    return emb(table, idx2)


def kernel(x, table):
    B0, B1 = x.shape
    B = B0 * B1
    D = table.shape[1]
    scale = float(D) ** 0.5
    idx2 = x.reshape(B // G, G).astype(jnp.int32)
    out = _emb_sc(table, idx2, B, D, scale)
    return out.reshape(B0, B1, D)


# speedup vs baseline: 1.2067x; 1.2067x over previous
import functools

import jax
import jax.numpy as jnp
from jax import lax
from jax.experimental import pallas as pl
from jax.experimental.pallas import tpu as pltpu
from jax.experimental.pallas import tpu_sc as plsc

LANES = 16
G = 128
NBUF = 4
LOOK = 2


def _emb_sc(table, idx2, B, D, scale):
    info = plsc.get_sparse_core_info()
    NC, NS = info.num_cores, info.num_subcores
    NW = NC * NS
    n_groups = idx2.shape[0]
    g_per_w = n_groups // NW
    n_outer = g_per_w // NBUF

    mesh = plsc.VectorSubcoreMesh(core_axis_name="c", subcore_axis_name="s")

    @functools.partial(
        pl.kernel,
        mesh=mesh,
        compiler_params=pltpu.CompilerParams(use_tc_tiling_on_sc=False),
        out_type=jax.ShapeDtypeStruct((B, D), jnp.float32),
        scratch_types=[
            pltpu.VMEM((g_per_w, G), jnp.int32),
            pltpu.VMEM((NBUF, G, D), jnp.float32),
            [pltpu.SemaphoreType.DMA] * NBUF,
            [pltpu.SemaphoreType.DMA] * NBUF,
        ],
    )
    def emb(table_hbm, idx_hbm, out_hbm, idx_v, rows_v, gsems, wsems):
        c = lax.axis_index("c")
        s = lax.axis_index("s")
        wid = s * NC + c
        gbase = wid * g_per_w
        pltpu.sync_copy(idx_hbm.at[pl.ds(gbase, g_per_w)], idx_v)

        def fire_gather(b, g):
            pltpu.async_copy(table_hbm.at[idx_v.at[g]], rows_v.at[b], gsems[b])

        def wait_gather(b, g):
            pltpu.make_async_copy(
                table_hbm.at[idx_v.at[g]], rows_v.at[b], gsems[b]).wait()

        def fire_write(b, g):
            pltpu.async_copy(
                rows_v.at[b], out_hbm.at[pl.ds((gbase + g) * G, G)], wsems[b])

        def wait_write(b, g):
            pltpu.make_async_copy(
                rows_v.at[b], out_hbm.at[pl.ds((gbase + g) * G, G)],
                wsems[b]).wait()

        for b in range(LOOK):
            fire_gather(b, b)

        def outer(o, carry):
            for b in range(NBUF):
                i = o * NBUF + b
                bf = (b + LOOK) % NBUF
                pf = i + LOOK

                @pl.when(pf < g_per_w)
                def _():
                    @pl.when(i >= LOOK)
                    def _():
                        wait_write(bf, i - LOOK)
                    fire_gather(bf, pf)

                wait_gather(b, i)

                def scale_row(r, carry2):
                    for j in range(D // LANES):
                        sl = pl.ds(j * LANES, LANES)
                        rows_v[b, r, sl] = rows_v[b, r, sl] * scale
                    return carry2

                lax.fori_loop(0, G, scale_row, 0, unroll=8)
                fire_write(b, i)
            return carry

        lax.fori_loop(0, n_outer, outer, 0)

        for b in range(NBUF):
            wait_write(b, g_per_w - NBUF + b)

    return emb(table, idx2)


def kernel(x, table):
    B0, B1 = x.shape
    B = B0 * B1
    D = table.shape[1]
    scale = float(D) ** 0.5
    idx2 = x.reshape(B // G, G).astype(jnp.int32)
    out = _emb_sc(table, idx2, B, D, scale)
    return out.reshape(B0, B1, D)


# speedup vs baseline: 1.3319x; 1.1037x over previous
import functools

import jax
import jax.numpy as jnp
import numpy as np
from jax import lax
from jax.experimental import pallas as pl
from jax.experimental.pallas import tpu as pltpu
from jax.experimental.pallas import tpu_sc as plsc

LANES = 16
G = 128
NBUF = 4
LOOK = 2
PITCH = 129


def _emb_sc(table, xt, n_b1, D):
    scale = float(D) ** 0.5
    info = plsc.get_sparse_core_info()
    NC, NS = info.num_cores, info.num_subcores
    NW = NC * NS
    DB = D // 8

    mesh = plsc.VectorSubcoreMesh(core_axis_name="c", subcore_axis_name="s")

    @functools.partial(
        pl.kernel,
        mesh=mesh,
        compiler_params=pltpu.CompilerParams(
            use_tc_tiling_on_sc=False, needs_layout_passes=False),
        out_type=jax.ShapeDtypeStruct((n_b1, DB, NW, 8, G), jnp.float32),
        scratch_types=[
            pltpu.VMEM((n_b1, G), jnp.int32),
            pltpu.VMEM((NBUF, G, D), jnp.float32),
            pltpu.VMEM((NBUF, D, PITCH), jnp.float32),
            [pltpu.SemaphoreType.DMA] * NBUF,
            [pltpu.SemaphoreType.DMA] * NBUF,
        ],
    )
    def emb(table_hbm, xt_hbm, out_hbm, idx_v, rows_v, tbuf, gsems, wsems):
        c_ax = lax.axis_index("c")
        s_ax = lax.axis_index("s")
        w = s_ax * NC + c_ax
        pltpu.sync_copy(xt_hbm.at[:, pl.ds(w * G, G)], idx_v)

        def fire_gather(b, g):
            pltpu.async_copy(table_hbm.at[idx_v.at[g]], rows_v.at[b], gsems[b])

        def wait_gather(b, g):
            pltpu.make_async_copy(
                table_hbm.at[idx_v.at[g]], rows_v.at[b], gsems[b]).wait()

        def fire_writes(b, g):
            for k in range(DB):
                pltpu.async_copy(
                    tbuf.at[b, pl.ds(k * 8, 8), pl.ds(0, G)],
                    out_hbm.at[g, k, w], wsems[b])

        def wait_writes(b, g):
            for k in range(DB):
                pltpu.make_async_copy(
                    tbuf.at[b, pl.ds(k * 8, 8), pl.ds(0, G)],
                    out_hbm.at[g, k, w], wsems[b]).wait()

        base_iota = lax.iota(jnp.int32, LANES)
        row_idx = [base_iota + d0 for d0 in range(0, D, LANES)]

        for b in range(LOOK):
            fire_gather(b, b)

        def outer(o, carry):
            for b in range(NBUF):
                g = o * NBUF + b
                bf = (b + LOOK) % NBUF
                pf = g + LOOK

                @pl.when(pf < n_b1)
                def _():
                    @pl.when(g >= LOOK)
                    def _():
                        wait_writes(bf, g - LOOK)
                    fire_gather(bf, pf)

                wait_gather(b, g)

                def col_body(c, carry2):
                    col = jnp.full((LANES,), c, dtype=jnp.int32)
                    for j in range(D // LANES):
                        vals = rows_v[b, c, pl.ds(j * LANES, LANES)] * scale
                        plsc.store_scatter(
                            tbuf.at[b], [row_idx[j], col], vals)
                    return carry2

                lax.fori_loop(0, G, col_body, 0, unroll=8)
                fire_writes(b, g)
            return carry

        lax.fori_loop(0, n_b1 // NBUF, outer, 0)

        for b in range(NBUF):
            wait_writes(b, n_b1 - NBUF + b)

    return emb(table, xt)


def kernel(x, table):
    B0, B1 = x.shape
    D = table.shape[1]
    xt = jnp.swapaxes(x, 0, 1).astype(jnp.int32)
    out5 = _emb_sc(table, xt, B1, D)
    return out5.transpose(2, 4, 0, 1, 3).reshape(B0, B1, D)


# speedup vs baseline: 1.9668x; 1.4767x over previous
import functools

import jax
import jax.numpy as jnp
import numpy as np
from jax import lax
from jax.experimental import pallas as pl
from jax.experimental.pallas import tpu as pltpu
from jax.experimental.pallas import tpu_sc as plsc

LANES = 16
G = 128
NBUF = 4
LOOK = 2
PITCH = 129


def _emb_sc(table, xt, n_b1, D):
    scale = float(D) ** 0.5
    info = plsc.get_sparse_core_info()
    NC, NS = info.num_cores, info.num_subcores
    NW = NC * NS
    DB = D // 8

    mesh = plsc.VectorSubcoreMesh(core_axis_name="c", subcore_axis_name="s")

    @functools.partial(
        pl.kernel,
        mesh=mesh,
        compiler_params=pltpu.CompilerParams(
            use_tc_tiling_on_sc=False, needs_layout_passes=False),
        out_type=jax.ShapeDtypeStruct((n_b1, DB, NW, 8, G), jnp.float32),
        scratch_types=[
            pltpu.VMEM((n_b1, G), jnp.int32),
            pltpu.VMEM((NBUF, G, D), jnp.float32),
            pltpu.VMEM((NBUF, D, PITCH), jnp.float32),
            [pltpu.SemaphoreType.DMA] * NBUF,
            [pltpu.SemaphoreType.DMA] * NBUF,
        ],
    )
    def emb(table_hbm, xt_hbm, out_hbm, idx_v, rows_v, tbuf, gsems, wsems):
        c_ax = lax.axis_index("c")
        s_ax = lax.axis_index("s")
        w = s_ax * NC + c_ax
        pltpu.sync_copy(xt_hbm.at[:, pl.ds(w * G, G)], idx_v)

        def fire_gather(b, g):
            pltpu.async_copy(table_hbm.at[idx_v.at[g]], rows_v.at[b], gsems[b])

        def wait_gather(b, g):
            pltpu.make_async_copy(
                table_hbm.at[idx_v.at[g]], rows_v.at[b], gsems[b]).wait()

        def fire_writes(b, g):
            for k in range(DB):
                pltpu.async_copy(
                    tbuf.at[b, pl.ds(k * 8, 8), pl.ds(0, G)],
                    out_hbm.at[g, k, w], wsems[b])

        def wait_writes(b, g):
            for k in range(DB):
                pltpu.make_async_copy(
                    tbuf.at[b, pl.ds(k * 8, 8), pl.ds(0, G)],
                    out_hbm.at[g, k, w], wsems[b]).wait()

        base_iota = lax.iota(jnp.int32, LANES)
        row_idx = [base_iota + d0 for d0 in range(0, D, LANES)]

        for b in range(LOOK):
            fire_gather(b, b)

        def outer(o, carry):
            for b in range(NBUF):
                g = o * NBUF + b
                bf = (b + LOOK) % NBUF
                pf = g + LOOK

                @pl.when(pf < n_b1)
                def _():
                    @pl.when(g >= LOOK)
                    def _():
                        wait_writes(bf, g - LOOK)
                    fire_gather(bf, pf)

                wait_gather(b, g)

                @plsc.parallel_loop(0, G, unroll=8)
                def col_body(c):
                    col = jnp.full((LANES,), c, dtype=jnp.int32)
                    for j in range(D // LANES):
                        vals = rows_v[b, c, pl.ds(j * LANES, LANES)] * scale
                        plsc.store_scatter(
                            tbuf.at[b], [row_idx[j], col], vals)
                fire_writes(b, g)
            return carry

        lax.fori_loop(0, n_b1 // NBUF, outer, 0)

        for b in range(NBUF):
            wait_writes(b, n_b1 - NBUF + b)

    return emb(table, xt)


def kernel(x, table):
    B0, B1 = x.shape
    D = table.shape[1]
    xt = jnp.swapaxes(x, 0, 1).astype(jnp.int32)
    out5 = _emb_sc(table, xt, B1, D)
    return out5.transpose(2, 4, 0, 1, 3).reshape(B0, B1, D)
